# Initial kernel scaffold; baseline (speedup 1.0000x reference)
import jax
import jax.numpy as jnp
from jax.experimental import pallas as pl


def kernel(feature_map, text_features, boxes, labels, whwh, W_vis, ln_g, ln_b, prototype_bank):
    raise NotImplementedError("write your pallas kernel here")


# speedup vs baseline: 2.1179x; 2.1179x over previous
import jax
import jax.numpy as jnp
from jax.experimental import pallas as pl

_C = 256
_D = 512
_K = 1000
_H = 64
_W = 64
_NB = 16
_B = 8
_BLEND = 0.35
_CTX_BLEND = 0.25
_HIGH = jax.lax.Precision.HIGHEST


def _pool_kernel(feat_ref, boxes_ref, whwh_ref, pooled_ref, gpool_ref):
    feat = feat_ref[0]
    bx = boxes_ref[0]
    wh = whwh_ref[0]
    img_w = jnp.maximum(wh[0:1, 0:1], 1.0)
    img_h = jnp.maximum(wh[0:1, 1:2], 1.0)
    scaled = bx * wh
    x1 = jnp.clip(jnp.floor(scaled[:, 0:1] / img_w * _W), 0.0, _W - 1.0)
    y1 = jnp.clip(jnp.floor(scaled[:, 1:2] / img_h * _H), 0.0, _H - 1.0)
    x2 = jnp.maximum(x1 + 1.0, jnp.minimum(float(_W), jnp.ceil(scaled[:, 2:3] / img_w * _W)))
    y2 = jnp.maximum(y1 + 1.0, jnp.minimum(float(_H), jnp.ceil(scaled[:, 3:4] / img_h * _H)))
    p = jax.lax.broadcasted_iota(jnp.int32, (_NB, _H * _W), 1)
    ym = (p // _W).astype(jnp.float32)
    xm = (p % _W).astype(jnp.float32)
    mask = ((ym >= y1) & (ym < y2) & (xm >= x1) & (xm < x2)).astype(jnp.float32)
    sums = jax.lax.dot_general(mask, feat, (((1,), (1,)), ((), ())),
                               preferred_element_type=jnp.float32,
                               precision=_HIGH)
    area = (x2 - x1) * (y2 - y1)
    pooled_ref[0] = sums / jnp.maximum(area, 1.0)
    gpool_ref[0] = jnp.mean(feat, axis=1, keepdims=True)


def _ctx_kernel(pooled_ref, gpool_ref, w_ref, g_ref, b_ref, labels_ref,
                bank_ref, ctx_ref):
    x = jnp.concatenate([pooled_ref[...], gpool_ref[...]], axis=0)
    h = jax.lax.dot_general(x, w_ref[...], (((1,), (0,)), ((), ())),
                            preferred_element_type=jnp.float32,
                            precision=_HIGH)
    mu = jnp.mean(h, axis=-1, keepdims=True)
    var = jnp.mean((h - mu) ** 2, axis=-1, keepdims=True)
    hn = (h - mu) / jnp.sqrt(var + 1e-5) * g_ref[...] + b_ref[...]
    nrm = jnp.sqrt(jnp.sum(hn * hn, axis=-1, keepdims=True))
    v = hn / jnp.maximum(nrm, 1e-6)
    vis = v[:_B * _NB]
    proj = v[_B * _NB:]

    labels = labels_ref[...]
    cls_iota = jax.lax.broadcasted_iota(jnp.int32, (_B * _NB, _K), 1)
    onehot = (labels == cls_iota).astype(jnp.float32)
    ones_col = jnp.ones((_B * _NB, 1), jnp.float32)
    cnts = jax.lax.dot_general(onehot, ones_col, (((0,), (0,)), ((), ())),
                               preferred_element_type=jnp.float32,
                               precision=_HIGH)
    sums = jax.lax.dot_general(onehot, vis, (((0,), (0,)), ((), ())),
                               preferred_element_type=jnp.float32,
                               precision=_HIGH)
    cls_mean = sums / jnp.maximum(cnts, 1.0)
    cn = jnp.sqrt(jnp.sum(cls_mean * cls_mean, axis=-1, keepdims=True))
    updated = cls_mean / jnp.maximum(cn, 1e-6)
    bank_new = jnp.where(cnts > 0.0, updated, bank_ref[...])

    logits = jax.lax.dot_general(proj, bank_new, (((1,), (1,)), ((), ())),
                                 preferred_element_type=jnp.float32,
                                 precision=_HIGH)
    m = jnp.max(logits, axis=-1, keepdims=True)
    e = jnp.exp(logits - m)
    wts = e / jnp.sum(e, axis=-1, keepdims=True)

    box_img = jax.lax.broadcasted_iota(jnp.int32, (_B, _B * _NB), 1) // _NB
    img_sel = (box_img == jax.lax.broadcasted_iota(jnp.int32, (_B, _B * _NB), 0)
               ).astype(jnp.float32)
    img_cnt = jax.lax.dot_general(img_sel, onehot, (((1,), (0,)), ((), ())),
                                  preferred_element_type=jnp.float32,
                                  precision=_HIGH)
    coeff = jnp.concatenate([wts, img_cnt * (1.0 / _NB)], axis=0)
    ctxs = jax.lax.dot_general(coeff, bank_new, (((1,), (0,)), ((), ())),
                               preferred_element_type=jnp.float32,
                               precision=_HIGH)
    sim_ctx = ctxs[:_B]
    label_ctx = ctxs[_B:]
    ctx_ref[...] = (1.0 - _CTX_BLEND) * label_ctx + _CTX_BLEND * sim_ctx


def _fuse_kernel(text_ref, ctx_ref, out_ref):
    t = text_ref[...]
    c = ctx_ref[...]
    fused = (1.0 - _BLEND) * t + _BLEND * c
    n = jnp.sqrt(jnp.sum(fused * fused, axis=-1, keepdims=True))
    out_ref[...] = fused / jnp.maximum(n, 1e-6)


def kernel(feature_map, text_features, boxes, labels, whwh, W_vis, ln_g, ln_b,
           prototype_bank):
    pooled, gpool = pl.pallas_call(
        _pool_kernel,
        grid=(_B,),
        in_specs=[
            pl.BlockSpec((1, _C, _H * _W), lambda i: (i, 0, 0)),
            pl.BlockSpec((1, _NB, 4), lambda i: (i, 0, 0)),
            pl.BlockSpec((1, 1, 4), lambda i: (i, 0, 0)),
        ],
        out_specs=[
            pl.BlockSpec((1, _NB, _C), lambda i: (i, 0, 0)),
            pl.BlockSpec((1, _C, 1), lambda i: (i, 0, 0)),
        ],
        out_shape=[
            jax.ShapeDtypeStruct((_B, _NB, _C), jnp.float32),
            jax.ShapeDtypeStruct((_B, _C, 1), jnp.float32),
        ],
    )(feature_map.reshape(_B, _C, _H * _W), boxes, whwh.reshape(_B, 1, 4))

    ctx = pl.pallas_call(
        _ctx_kernel,
        out_shape=jax.ShapeDtypeStruct((_B, _D), jnp.float32),
    )(pooled.reshape(_B * _NB, _C), gpool.reshape(_B, _C), W_vis,
      ln_g.reshape(1, _D), ln_b.reshape(1, _D),
      labels.reshape(_B * _NB, 1), prototype_bank)

    cb = 200
    out = pl.pallas_call(
        _fuse_kernel,
        grid=(_K // cb,),
        in_specs=[
            pl.BlockSpec((1, cb, _D), lambda i: (0, i, 0)),
            pl.BlockSpec((_B, 1, _D), lambda i: (0, 0, 0)),
        ],
        out_specs=pl.BlockSpec((_B, cb, _D), lambda i: (0, i, 0)),
        out_shape=jax.ShapeDtypeStruct((_B, _K, _D), jnp.float32),
    )(text_features.reshape(1, _K, _D), ctx.reshape(_B, 1, _D))
    return out


# speedup vs baseline: 2.8634x; 1.3520x over previous
import jax
import jax.numpy as jnp
from jax.experimental import pallas as pl

_C = 256
_D = 512
_K = 1000
_H = 64
_W = 64
_NB = 16
_B = 8
_BLEND = 0.35
_CTX_BLEND = 0.25
_HIGH = jax.lax.Precision.DEFAULT


def _pool_kernel(feat_ref, boxes_ref, whwh_ref, pooled_ref, gpool_ref):
    feat = feat_ref[0]
    bx = boxes_ref[0]
    wh = whwh_ref[0]
    img_w = jnp.maximum(wh[0:1, 0:1], 1.0)
    img_h = jnp.maximum(wh[0:1, 1:2], 1.0)
    scaled = bx * wh
    x1 = jnp.clip(jnp.floor(scaled[:, 0:1] / img_w * _W), 0.0, _W - 1.0)
    y1 = jnp.clip(jnp.floor(scaled[:, 1:2] / img_h * _H), 0.0, _H - 1.0)
    x2 = jnp.maximum(x1 + 1.0, jnp.minimum(float(_W), jnp.ceil(scaled[:, 2:3] / img_w * _W)))
    y2 = jnp.maximum(y1 + 1.0, jnp.minimum(float(_H), jnp.ceil(scaled[:, 3:4] / img_h * _H)))
    p = jax.lax.broadcasted_iota(jnp.int32, (_NB, _H * _W), 1)
    ym = (p // _W).astype(jnp.float32)
    xm = (p % _W).astype(jnp.float32)
    mask = ((ym >= y1) & (ym < y2) & (xm >= x1) & (xm < x2)).astype(jnp.float32)
    sums = jax.lax.dot_general(mask, feat, (((1,), (1,)), ((), ())),
                               preferred_element_type=jnp.float32,
                               precision=_HIGH)
    area = (x2 - x1) * (y2 - y1)
    pooled_ref[0] = sums / jnp.maximum(area, 1.0)
    gpool_ref[0] = jnp.mean(feat, axis=1, keepdims=True)


def _ctx_kernel(pooled_ref, gpool_ref, w_ref, g_ref, b_ref, labels_ref,
                bank_ref, ctx_ref):
    x = jnp.concatenate([pooled_ref[...], gpool_ref[...]], axis=0)
    h = jax.lax.dot_general(x, w_ref[...], (((1,), (0,)), ((), ())),
                            preferred_element_type=jnp.float32,
                            precision=_HIGH)
    mu = jnp.mean(h, axis=-1, keepdims=True)
    var = jnp.mean((h - mu) ** 2, axis=-1, keepdims=True)
    hn = (h - mu) / jnp.sqrt(var + 1e-5) * g_ref[...] + b_ref[...]
    nrm = jnp.sqrt(jnp.sum(hn * hn, axis=-1, keepdims=True))
    v = hn / jnp.maximum(nrm, 1e-6)
    vis = v[:_B * _NB]
    proj = v[_B * _NB:]

    labels = labels_ref[...]
    cls_iota = jax.lax.broadcasted_iota(jnp.int32, (_B * _NB, _K), 1)
    onehot = (labels == cls_iota).astype(jnp.float32)
    ones_col = jnp.ones((_B * _NB, 1), jnp.float32)
    cnts = jax.lax.dot_general(onehot, ones_col, (((0,), (0,)), ((), ())),
                               preferred_element_type=jnp.float32,
                               precision=_HIGH)
    sums = jax.lax.dot_general(onehot, vis, (((0,), (0,)), ((), ())),
                               preferred_element_type=jnp.float32,
                               precision=_HIGH)
    cls_mean = sums / jnp.maximum(cnts, 1.0)
    cn = jnp.sqrt(jnp.sum(cls_mean * cls_mean, axis=-1, keepdims=True))
    updated = cls_mean / jnp.maximum(cn, 1e-6)
    bank_new = jnp.where(cnts > 0.0, updated, bank_ref[...])

    logits = jax.lax.dot_general(proj, bank_new, (((1,), (1,)), ((), ())),
                                 preferred_element_type=jnp.float32,
                                 precision=_HIGH)
    m = jnp.max(logits, axis=-1, keepdims=True)
    e = jnp.exp(logits - m)
    wts = e / jnp.sum(e, axis=-1, keepdims=True)

    box_img = jax.lax.broadcasted_iota(jnp.int32, (_B, _B * _NB), 1) // _NB
    img_sel = (box_img == jax.lax.broadcasted_iota(jnp.int32, (_B, _B * _NB), 0)
               ).astype(jnp.float32)
    img_cnt = jax.lax.dot_general(img_sel, onehot, (((1,), (0,)), ((), ())),
                                  preferred_element_type=jnp.float32,
                                  precision=_HIGH)
    coeff = jnp.concatenate([wts, img_cnt * (1.0 / _NB)], axis=0)
    ctxs = jax.lax.dot_general(coeff, bank_new, (((1,), (0,)), ((), ())),
                               preferred_element_type=jnp.float32,
                               precision=_HIGH)
    sim_ctx = ctxs[:_B]
    label_ctx = ctxs[_B:]
    ctx_ref[...] = (1.0 - _CTX_BLEND) * label_ctx + _CTX_BLEND * sim_ctx


def _fuse_kernel(text_ref, ctx_ref, out_ref):
    t = text_ref[...]
    c = ctx_ref[...]
    fused = (1.0 - _BLEND) * t + _BLEND * c
    n = jnp.sqrt(jnp.sum(fused * fused, axis=-1, keepdims=True))
    out_ref[...] = fused / jnp.maximum(n, 1e-6)


def kernel(feature_map, text_features, boxes, labels, whwh, W_vis, ln_g, ln_b,
           prototype_bank):
    pooled, gpool = pl.pallas_call(
        _pool_kernel,
        grid=(_B,),
        in_specs=[
            pl.BlockSpec((1, _C, _H * _W), lambda i: (i, 0, 0)),
            pl.BlockSpec((1, _NB, 4), lambda i: (i, 0, 0)),
            pl.BlockSpec((1, 1, 4), lambda i: (i, 0, 0)),
        ],
        out_specs=[
            pl.BlockSpec((1, _NB, _C), lambda i: (i, 0, 0)),
            pl.BlockSpec((1, _C, 1), lambda i: (i, 0, 0)),
        ],
        out_shape=[
            jax.ShapeDtypeStruct((_B, _NB, _C), jnp.float32),
            jax.ShapeDtypeStruct((_B, _C, 1), jnp.float32),
        ],
    )(feature_map.reshape(_B, _C, _H * _W), boxes, whwh.reshape(_B, 1, 4))

    ctx = pl.pallas_call(
        _ctx_kernel,
        out_shape=jax.ShapeDtypeStruct((_B, _D), jnp.float32),
    )(pooled.reshape(_B * _NB, _C), gpool.reshape(_B, _C), W_vis,
      ln_g.reshape(1, _D), ln_b.reshape(1, _D),
      labels.reshape(_B * _NB, 1), prototype_bank)

    cb = 200
    out = pl.pallas_call(
        _fuse_kernel,
        grid=(_K // cb,),
        in_specs=[
            pl.BlockSpec((1, cb, _D), lambda i: (0, i, 0)),
            pl.BlockSpec((_B, 1, _D), lambda i: (0, 0, 0)),
        ],
        out_specs=pl.BlockSpec((_B, cb, _D), lambda i: (0, i, 0)),
        out_shape=jax.ShapeDtypeStruct((_B, _K, _D), jnp.float32),
    )(text_features.reshape(1, _K, _D), ctx.reshape(_B, 1, _D))
    return out
